# baseline (speedup 1.0000x reference)
import jax
import jax.numpy as jnp
from jax import lax
from jax.experimental import pallas as pl
from jax.experimental.pallas import tpu as pltpu
from jax.experimental.pallas import tpu_sc as plsc

N = 10000
NP = 10240
E = 160000
DC = 128
NC = 2
NS = 16
C = 80
K = 4
SUPC = K * C
E_PAD = 163840
NROW = E_PAD // SUPC
SUP_T = NROW // NS
SUP_DT = NROW // (NC * NS)
RPT = NP // NS


def _fill3d(ref, val):
    d0, d1, d2 = ref.shape
    nv = d2 // 16
    vec = jnp.full((16,), val, ref.dtype)

    def body(i, carry):
        g = i // (d1 * nv)
        rem = i - g * (d1 * nv)
        r = rem // nv
        j = rem - r * nv
        ref[g, r, pl.ds(j * 16, 16)] = vec
        return carry

    lax.fori_loop(0, d0 * d1 * nv, body, 0)


def _make_sc_agg(q_chunks, with_deg):
    mesh = plsc.VectorSubcoreMesh(core_axis_name="c", subcore_axis_name="s")
    out_type = [jax.ShapeDtypeStruct((q_chunks * NP, DC), jnp.float32)]
    if with_deg:
        out_type.append(jax.ShapeDtypeStruct((NC * NP, DC), jnp.float32))
    scratch = [
        pltpu.VMEM((K, C), jnp.int32),
        pltpu.VMEM((K, C), jnp.int32),
        pltpu.VMEM((K, C, DC), jnp.float32),
        pltpu.MemorySpace.VMEM_SHARED((NP, DC), jnp.float32),
        pltpu.SemaphoreType.DMA,
        pltpu.SemaphoreType.DMA,
    ]

    def body(table, src3d, dst3d, zeros, agg_out, *rest):
        if with_deg:
            deg_out = rest[0]
            rest = rest[1:]
        src_big, dst_big, rows, acc, gsem, ssem = rest
        c = lax.axis_index("c")
        s = lax.axis_index("s")
        r0 = s * RPT

        n_passes = q_chunks // NC
        for p in range(n_passes):
            qn = (p * NC + c) * NP

            pltpu.sync_copy(zeros, acc.at[pl.ds(r0, RPT)])
            plsc.subcore_barrier()

            def edge_body(i, carry):
                row0 = s * SUP_T + i
                pltpu.sync_copy(src3d.at[row0], src_big)
                pltpu.sync_copy(dst3d.at[row0], dst_big)
                for g in range(K):
                    for j in range(C // 16):
                        sl = pl.ds(j * 16, 16)
                        src_big[g, sl] = src_big[g, sl] + qn
                copies = [pltpu.async_copy(table.at[src_big.at[g]],
                                           rows.at[g], gsem)
                          for g in range(K)]
                for cp in copies:
                    cp.wait()
                adds = [pltpu.async_copy(rows.at[g], acc.at[dst_big.at[g]],
                                         ssem, add=True)
                        for g in range(K)]
                for cp in adds:
                    cp.wait()
                return carry

            lax.fori_loop(0, SUP_T, edge_body, 0)
            plsc.subcore_barrier()

            pltpu.sync_copy(acc.at[pl.ds(r0, RPT)],
                            agg_out.at[pl.ds(qn + r0, RPT)])
            if p + 1 < n_passes or with_deg:
                plsc.subcore_barrier()

        if with_deg:
            pltpu.sync_copy(zeros, acc.at[pl.ds(r0, RPT)])
            _fill3d(rows, 1.0)
            plsc.subcore_barrier()

            def deg_body(i, carry):
                row0 = c * (NROW // NC) + s * SUP_DT + i
                pltpu.sync_copy(dst3d.at[row0], dst_big)
                adds = [pltpu.async_copy(rows.at[g], acc.at[dst_big.at[g]],
                                         ssem, add=True)
                        for g in range(K)]
                for cp in adds:
                    cp.wait()
                return carry

            lax.fori_loop(0, SUP_DT, deg_body, 0)
            plsc.subcore_barrier()
            pltpu.sync_copy(acc.at[pl.ds(r0, RPT)],
                            deg_out.at[pl.ds(c * NP + r0, RPT)])

    return pl.kernel(body, out_type=out_type, mesh=mesh, scratch_types=scratch)


BN = 640


def _k1_body(x_ref, a0_ref, a1_ref, d0_ref, d1_ref, ws_ref, wn_ref, b_ref,
             o_ref):
    agg = jnp.concatenate([a0_ref[...], a1_ref[...]], axis=1)
    deg = d0_ref[...][:, 0:1] + d1_ref[...][:, 0:1]
    recip = 1.0 / jnp.maximum(deg, 1.0)
    acc = jnp.dot(x_ref[...], ws_ref[...], preferred_element_type=jnp.float32)
    acc = acc + jnp.dot(agg, wn_ref[...], preferred_element_type=jnp.float32) * recip
    o_ref[...] = jnp.maximum(acc + b_ref[...], 0.0)


def _layer1(x, agg1c, deg2, w_self, w_neigh, b):
    h = x.shape[1]
    nb = NP // BN
    return pl.pallas_call(
        _k1_body,
        grid=(nb, 4),
        in_specs=[
            pl.BlockSpec((BN, h), lambda i, q: (i, 0)),
            pl.BlockSpec((BN, DC), lambda i, q: (i, 0)),
            pl.BlockSpec((BN, DC), lambda i, q: (nb + i, 0)),
            pl.BlockSpec((BN, DC), lambda i, q: (i, 0)),
            pl.BlockSpec((BN, DC), lambda i, q: (nb + i, 0)),
            pl.BlockSpec((h, DC), lambda i, q: (0, q)),
            pl.BlockSpec((h, DC), lambda i, q: (0, q)),
            pl.BlockSpec((1, DC), lambda i, q: (0, q)),
        ],
        out_specs=pl.BlockSpec((BN, DC), lambda i, q: (q * nb + i, 0)),
        out_shape=jax.ShapeDtypeStruct((4 * NP, DC), jnp.float32),
    )(x, agg1c, agg1c, deg2, deg2, w_self, w_neigh, b)


def _k2_body(h0, h1, h2, h3, a0, a1, a2, a3, d0_ref, d1_ref, ws_ref, wn_ref,
             b2_ref, wf1_ref, bf1_ref, wf2_ref, bf2_ref, o_ref):
    h = jnp.concatenate([h0[...], h1[...], h2[...], h3[...]], axis=1)
    agg = jnp.concatenate([a0[...], a1[...], a2[...], a3[...]], axis=1)
    deg = d0_ref[...][:, 0:1] + d1_ref[...][:, 0:1]
    recip = 1.0 / jnp.maximum(deg, 1.0)
    acc = jnp.dot(h, ws_ref[...], preferred_element_type=jnp.float32)
    acc = acc + jnp.dot(agg, wn_ref[...], preferred_element_type=jnp.float32) * recip
    h2v = jnp.maximum(acc + b2_ref[...], 0.0)
    z = jnp.maximum(
        jnp.dot(h2v, wf1_ref[...], preferred_element_type=jnp.float32) + bf1_ref[...],
        0.0)
    o_ref[...] = (jnp.dot(z, wf2_ref[...], preferred_element_type=jnp.float32)
                  + bf2_ref[...])


def _layer2_head(h1c, agg2c, deg2, w_self, w_neigh, b2, w_fc1, b_fc1, w_fc2,
                 b_fc2):
    nb = NP // BN
    hh = w_self.shape[0]
    chunk_spec = lambda q: pl.BlockSpec((BN, DC), lambda i, q=q: (q * nb + i, 0))
    return pl.pallas_call(
        _k2_body,
        grid=(nb,),
        in_specs=(
            [chunk_spec(q) for q in range(4)]
            + [chunk_spec(q) for q in range(4)]
            + [
                pl.BlockSpec((BN, DC), lambda i: (i, 0)),
                pl.BlockSpec((BN, DC), lambda i: (nb + i, 0)),
                pl.BlockSpec((hh, hh), lambda i: (0, 0)),
                pl.BlockSpec((hh, hh), lambda i: (0, 0)),
                pl.BlockSpec((1, hh), lambda i: (0, 0)),
                pl.BlockSpec((hh, hh), lambda i: (0, 0)),
                pl.BlockSpec((1, hh), lambda i: (0, 0)),
                pl.BlockSpec((hh, 1), lambda i: (0, 0)),
                pl.BlockSpec((1, 1), lambda i: (0, 0)),
            ]
        ),
        out_specs=pl.BlockSpec((BN, 1), lambda i: (i, 0)),
        out_shape=jax.ShapeDtypeStruct((NP, 1), jnp.float32),
    )(*([h1c] * 4), *([agg2c] * 4), deg2, deg2, w_self, w_neigh, b2, w_fc1,
      b_fc1, w_fc2, b_fc2)


def kernel(x, edge_index, W_self1, W_neigh1, b1, W_self2, W_neigh2, b2,
           W_fc1, b_fc1, W_fc2, b_fc2):
    src = edge_index[0]
    dst = edge_index[1]
    xp = jnp.pad(x, ((0, NP - N), (0, 0)))
    xc = xp.reshape(NP, 2, DC).transpose(1, 0, 2).reshape(2 * NP, DC)

    src3d = jnp.pad(src, (0, E_PAD - E), constant_values=N).reshape(NROW, K, C)
    dst3d = jnp.pad(dst, (0, E_PAD - E), constant_values=N).reshape(NROW, K, C)
    zeros = jnp.zeros((RPT, DC), jnp.float32)
    agg1c, deg2 = _make_sc_agg(2, True)(xc, src3d, dst3d, zeros)
    h1c = _layer1(xp, agg1c, deg2, W_self1, W_neigh1, b1.reshape(1, -1))
    (agg2c,) = _make_sc_agg(4, False)(h1c, src3d, dst3d, zeros)
    scores = _layer2_head(h1c, agg2c, deg2, W_self2, W_neigh2,
                          b2.reshape(1, -1), W_fc1, b_fc1.reshape(1, -1),
                          W_fc2, b_fc2.reshape(1, 1))
    return scores[:N]


# speedup vs baseline: 1.1804x; 1.1804x over previous
import jax
import jax.numpy as jnp
from jax import lax
from jax.experimental import pallas as pl
from jax.experimental.pallas import tpu as pltpu
from jax.experimental.pallas import tpu_sc as plsc

N = 10000
NP = 10240
E = 160000
DC = 128
NC = 2
NS = 16
C = 128
K = 8
SUPC = K * C
E_PAD = 163840
NROW = E_PAD // SUPC
SUP_T = NROW // NS
SUP_DT = NROW // (NC * NS)
RPT = NP // NS


def _fill3d(ref, val):
    d0, d1, d2 = ref.shape
    nv = d2 // 16
    vec = jnp.full((16,), val, ref.dtype)

    def body(i, carry):
        g = i // (d1 * nv)
        rem = i - g * (d1 * nv)
        r = rem // nv
        j = rem - r * nv
        ref[g, r, pl.ds(j * 16, 16)] = vec
        return carry

    lax.fori_loop(0, d0 * d1 * nv, body, 0)


def _make_sc_agg(q_chunks, with_deg):
    mesh = plsc.VectorSubcoreMesh(core_axis_name="c", subcore_axis_name="s")
    out_type = [jax.ShapeDtypeStruct((q_chunks * NP, DC), jnp.float32)]
    if with_deg:
        out_type.append(jax.ShapeDtypeStruct((NC * NP, DC), jnp.float32))
    scratch = [
        pltpu.VMEM((K, C), jnp.int32),
        pltpu.VMEM((K, C), jnp.int32),
        pltpu.VMEM((2, C, DC), jnp.float32),
        pltpu.MemorySpace.VMEM_SHARED((NP, DC), jnp.float32),
        pltpu.SemaphoreType.DMA,
        pltpu.SemaphoreType.DMA,
    ]

    def body(table, src3d, dst3d, zeros, agg_out, *rest):
        if with_deg:
            deg_out = rest[0]
            rest = rest[1:]
        src_big, dst_big, rows, acc, gsem, ssem = rest
        c = lax.axis_index("c")
        s = lax.axis_index("s")
        r0 = s * RPT

        n_passes = q_chunks // NC
        for p in range(n_passes):
            qn = (p * NC + c) * NP

            pltpu.sync_copy(zeros, acc.at[pl.ds(r0, RPT)])
            plsc.subcore_barrier()

            def edge_body(i, carry):
                row0 = s * SUP_T + i
                pltpu.sync_copy(src3d.at[row0], src_big)
                pltpu.sync_copy(dst3d.at[row0], dst_big)
                for g in range(K):
                    for j in range(C // 16):
                        sl = pl.ds(j * 16, 16)
                        src_big[g, sl] = src_big[g, sl] + qn
                gath = {0: pltpu.async_copy(table.at[src_big.at[0]],
                                            rows.at[0], gsem)}
                scat = {}
                for g in range(K):
                    if g + 1 < K:
                        if g >= 1:
                            scat[g - 1].wait()
                        gath[g + 1] = pltpu.async_copy(
                            table.at[src_big.at[g + 1]],
                            rows.at[(g + 1) % 2], gsem)
                    gath[g].wait()
                    scat[g] = pltpu.async_copy(rows.at[g % 2],
                                               acc.at[dst_big.at[g]],
                                               ssem, add=True)
                scat[K - 2].wait()
                scat[K - 1].wait()
                return carry

            lax.fori_loop(0, SUP_T, edge_body, 0)
            plsc.subcore_barrier()

            pltpu.sync_copy(acc.at[pl.ds(r0, RPT)],
                            agg_out.at[pl.ds(qn + r0, RPT)])
            if p + 1 < n_passes or with_deg:
                plsc.subcore_barrier()

        if with_deg:
            pltpu.sync_copy(zeros, acc.at[pl.ds(r0, RPT)])
            _fill3d(rows, 1.0)
            plsc.subcore_barrier()

            def deg_body(i, carry):
                row0 = c * (NROW // NC) + s * SUP_DT + i
                pltpu.sync_copy(dst3d.at[row0], dst_big)
                adds = [pltpu.async_copy(rows.at[0], acc.at[dst_big.at[g]],
                                         ssem, add=True)
                        for g in range(K)]
                for cp in adds:
                    cp.wait()
                return carry

            lax.fori_loop(0, SUP_DT, deg_body, 0)
            plsc.subcore_barrier()
            pltpu.sync_copy(acc.at[pl.ds(r0, RPT)],
                            deg_out.at[pl.ds(c * NP + r0, RPT)])

    return pl.kernel(body, out_type=out_type, mesh=mesh, scratch_types=scratch)


BN = 640


def _k1_body(x_ref, a0_ref, a1_ref, d0_ref, d1_ref, ws_ref, wn_ref, b_ref,
             o_ref):
    agg = jnp.concatenate([a0_ref[...], a1_ref[...]], axis=1)
    deg = d0_ref[...][:, 0:1] + d1_ref[...][:, 0:1]
    recip = 1.0 / jnp.maximum(deg, 1.0)
    acc = jnp.dot(x_ref[...], ws_ref[...], preferred_element_type=jnp.float32)
    acc = acc + jnp.dot(agg, wn_ref[...], preferred_element_type=jnp.float32) * recip
    o_ref[...] = jnp.maximum(acc + b_ref[...], 0.0)


def _layer1(x, agg1c, deg2, w_self, w_neigh, b):
    h = x.shape[1]
    nb = NP // BN
    return pl.pallas_call(
        _k1_body,
        grid=(nb, 4),
        in_specs=[
            pl.BlockSpec((BN, h), lambda i, q: (i, 0)),
            pl.BlockSpec((BN, DC), lambda i, q: (i, 0)),
            pl.BlockSpec((BN, DC), lambda i, q: (nb + i, 0)),
            pl.BlockSpec((BN, DC), lambda i, q: (i, 0)),
            pl.BlockSpec((BN, DC), lambda i, q: (nb + i, 0)),
            pl.BlockSpec((h, DC), lambda i, q: (0, q)),
            pl.BlockSpec((h, DC), lambda i, q: (0, q)),
            pl.BlockSpec((1, DC), lambda i, q: (0, q)),
        ],
        out_specs=pl.BlockSpec((BN, DC), lambda i, q: (q * nb + i, 0)),
        out_shape=jax.ShapeDtypeStruct((4 * NP, DC), jnp.float32),
    )(x, agg1c, agg1c, deg2, deg2, w_self, w_neigh, b)


def _k2_body(h0, h1, h2, h3, a0, a1, a2, a3, d0_ref, d1_ref, ws_ref, wn_ref,
             b2_ref, wf1_ref, bf1_ref, wf2_ref, bf2_ref, o_ref):
    h = jnp.concatenate([h0[...], h1[...], h2[...], h3[...]], axis=1)
    agg = jnp.concatenate([a0[...], a1[...], a2[...], a3[...]], axis=1)
    deg = d0_ref[...][:, 0:1] + d1_ref[...][:, 0:1]
    recip = 1.0 / jnp.maximum(deg, 1.0)
    acc = jnp.dot(h, ws_ref[...], preferred_element_type=jnp.float32)
    acc = acc + jnp.dot(agg, wn_ref[...], preferred_element_type=jnp.float32) * recip
    h2v = jnp.maximum(acc + b2_ref[...], 0.0)
    z = jnp.maximum(
        jnp.dot(h2v, wf1_ref[...], preferred_element_type=jnp.float32) + bf1_ref[...],
        0.0)
    o_ref[...] = (jnp.dot(z, wf2_ref[...], preferred_element_type=jnp.float32)
                  + bf2_ref[...])


def _layer2_head(h1c, agg2c, deg2, w_self, w_neigh, b2, w_fc1, b_fc1, w_fc2,
                 b_fc2):
    nb = NP // BN
    hh = w_self.shape[0]
    chunk_spec = lambda q: pl.BlockSpec((BN, DC), lambda i, q=q: (q * nb + i, 0))
    return pl.pallas_call(
        _k2_body,
        grid=(nb,),
        in_specs=(
            [chunk_spec(q) for q in range(4)]
            + [chunk_spec(q) for q in range(4)]
            + [
                pl.BlockSpec((BN, DC), lambda i: (i, 0)),
                pl.BlockSpec((BN, DC), lambda i: (nb + i, 0)),
                pl.BlockSpec((hh, hh), lambda i: (0, 0)),
                pl.BlockSpec((hh, hh), lambda i: (0, 0)),
                pl.BlockSpec((1, hh), lambda i: (0, 0)),
                pl.BlockSpec((hh, hh), lambda i: (0, 0)),
                pl.BlockSpec((1, hh), lambda i: (0, 0)),
                pl.BlockSpec((hh, 1), lambda i: (0, 0)),
                pl.BlockSpec((1, 1), lambda i: (0, 0)),
            ]
        ),
        out_specs=pl.BlockSpec((BN, 1), lambda i: (i, 0)),
        out_shape=jax.ShapeDtypeStruct((NP, 1), jnp.float32),
    )(*([h1c] * 4), *([agg2c] * 4), deg2, deg2, w_self, w_neigh, b2, w_fc1,
      b_fc1, w_fc2, b_fc2)


def kernel(x, edge_index, W_self1, W_neigh1, b1, W_self2, W_neigh2, b2,
           W_fc1, b_fc1, W_fc2, b_fc2):
    src = edge_index[0]
    dst = edge_index[1]
    xp = jnp.pad(x, ((0, NP - N), (0, 0)))
    xc = xp.reshape(NP, 2, DC).transpose(1, 0, 2).reshape(2 * NP, DC)

    src3d = jnp.pad(src, (0, E_PAD - E), constant_values=N).reshape(NROW, K, C)
    dst3d = jnp.pad(dst, (0, E_PAD - E), constant_values=N).reshape(NROW, K, C)
    zeros = jnp.zeros((RPT, DC), jnp.float32)
    agg1c, deg2 = _make_sc_agg(2, True)(xc, src3d, dst3d, zeros)
    h1c = _layer1(xp, agg1c, deg2, W_self1, W_neigh1, b1.reshape(1, -1))
    (agg2c,) = _make_sc_agg(4, False)(h1c, src3d, dst3d, zeros)
    scores = _layer2_head(h1c, agg2c, deg2, W_self2, W_neigh2,
                          b2.reshape(1, -1), W_fc1, b_fc1.reshape(1, -1),
                          W_fc2, b_fc2.reshape(1, 1))
    return scores[:N]


# speedup vs baseline: 1.2352x; 1.0464x over previous
import jax
import jax.numpy as jnp
from jax import lax
from jax.experimental import pallas as pl
from jax.experimental.pallas import tpu as pltpu
from jax.experimental.pallas import tpu_sc as plsc

N = 10000
NP = 10240
E = 160000
DC = 128
NC = 2
NS = 16
C = 128
K = 8
SUPC = K * C
E_PAD = 163840
NROW = E_PAD // SUPC
SUP_T = NROW // NS
SUP_DT = NROW // (NC * NS)
RPT = NP // NS


def _fill3d(ref, val):
    d0, d1, d2 = ref.shape
    nv = d2 // 16
    vec = jnp.full((16,), val, ref.dtype)

    def body(i, carry):
        g = i // (d1 * nv)
        rem = i - g * (d1 * nv)
        r = rem // nv
        j = rem - r * nv
        ref[g, r, pl.ds(j * 16, 16)] = vec
        return carry

    lax.fori_loop(0, d0 * d1 * nv, body, 0)


def _make_sc_agg(q_chunks, with_deg):
    mesh = plsc.VectorSubcoreMesh(core_axis_name="c", subcore_axis_name="s")
    out_type = [jax.ShapeDtypeStruct((q_chunks * NP, DC), jnp.float32)]
    if with_deg:
        out_type.append(jax.ShapeDtypeStruct((NC * NP, DC), jnp.float32))
    scratch = [
        pltpu.VMEM((2, K, C), jnp.int32),
        pltpu.VMEM((2, K, C), jnp.int32),
        pltpu.VMEM((2, C, DC), jnp.float32),
        pltpu.MemorySpace.VMEM_SHARED((NP, DC), jnp.float32),
        pltpu.SemaphoreType.DMA,
        pltpu.SemaphoreType.DMA,
        pltpu.SemaphoreType.DMA,
    ]

    def body(table, src3d, dst3d, zeros, agg_out, *rest):
        if with_deg:
            deg_out = rest[0]
            rest = rest[1:]
        src_big, dst_big, rows, acc, gsem, ssem, isem = rest
        c = lax.axis_index("c")
        s = lax.axis_index("s")
        r0 = s * RPT

        n_passes = q_chunks // NC
        for p in range(n_passes):
            qn = (p * NC + c) * NP

            pltpu.sync_copy(zeros, acc.at[pl.ds(r0, RPT)])
            plsc.subcore_barrier()

            pltpu.sync_copy(src3d.at[s * SUP_T], src_big.at[0])
            pltpu.sync_copy(dst3d.at[s * SUP_T], dst_big.at[0])

            def edge_body(i, carry):
                b = lax.rem(i, 2)
                nxt = 1 - b
                row_next = s * SUP_T + jnp.minimum(i + 1, SUP_T - 1)
                pf = [pltpu.async_copy(src3d.at[row_next], src_big.at[nxt],
                                       isem),
                      pltpu.async_copy(dst3d.at[row_next], dst_big.at[nxt],
                                       isem)]
                for g in range(K):
                    for j in range(C // 16):
                        sl = pl.ds(j * 16, 16)
                        src_big[b, g, sl] = src_big[b, g, sl] + qn
                gath = {0: pltpu.async_copy(table.at[src_big.at[b, 0]],
                                            rows.at[0], gsem)}
                scat = {}
                for g in range(K):
                    if g + 1 < K:
                        if g >= 1:
                            scat[g - 1].wait()
                        gath[g + 1] = pltpu.async_copy(
                            table.at[src_big.at[b, g + 1]],
                            rows.at[(g + 1) % 2], gsem)
                    gath[g].wait()
                    scat[g] = pltpu.async_copy(rows.at[g % 2],
                                               acc.at[dst_big.at[b, g]],
                                               ssem, add=True)
                scat[K - 2].wait()
                scat[K - 1].wait()
                for cp in pf:
                    cp.wait()
                return carry

            lax.fori_loop(0, SUP_T, edge_body, 0)
            plsc.subcore_barrier()

            pltpu.sync_copy(acc.at[pl.ds(r0, RPT)],
                            agg_out.at[pl.ds(qn + r0, RPT)])
            if p + 1 < n_passes or with_deg:
                plsc.subcore_barrier()

        if with_deg:
            pltpu.sync_copy(zeros, acc.at[pl.ds(r0, RPT)])
            _fill3d(rows, 1.0)
            plsc.subcore_barrier()

            d0 = c * (NROW // NC) + s * SUP_DT
            pltpu.sync_copy(dst3d.at[d0], dst_big.at[0])

            def deg_body(i, carry):
                b = lax.rem(i, 2)
                nxt = 1 - b
                row_next = d0 + jnp.minimum(i + 1, SUP_DT - 1)
                pf = pltpu.async_copy(dst3d.at[row_next], dst_big.at[nxt],
                                      isem)
                adds = [pltpu.async_copy(rows.at[0], acc.at[dst_big.at[b, g]],
                                         ssem, add=True)
                        for g in range(K)]
                for cp in adds:
                    cp.wait()
                pf.wait()
                return carry

            lax.fori_loop(0, SUP_DT, deg_body, 0)
            plsc.subcore_barrier()
            pltpu.sync_copy(acc.at[pl.ds(r0, RPT)],
                            deg_out.at[pl.ds(c * NP + r0, RPT)])

    return pl.kernel(body, out_type=out_type, mesh=mesh, scratch_types=scratch)


BN = 1280


def _k1_body(x_ref, a0_ref, a1_ref, d0_ref, d1_ref, ws_ref, wn_ref, b_ref,
             o_ref):
    agg = jnp.concatenate([a0_ref[...], a1_ref[...]], axis=1)
    deg = d0_ref[...][:, 0:1] + d1_ref[...][:, 0:1]
    recip = 1.0 / jnp.maximum(deg, 1.0)
    acc = jnp.dot(x_ref[...].astype(jnp.bfloat16), ws_ref[...],
                  preferred_element_type=jnp.float32)
    acc = acc + jnp.dot(agg.astype(jnp.bfloat16), wn_ref[...],
                        preferred_element_type=jnp.float32) * recip
    o_ref[...] = jnp.maximum(acc + b_ref[...], 0.0)


def _layer1(x, agg1c, deg2, w_self, w_neigh, b):
    h = x.shape[1]
    nb = NP // BN
    return pl.pallas_call(
        _k1_body,
        grid=(nb, 4),
        in_specs=[
            pl.BlockSpec((BN, h), lambda i, q: (i, 0)),
            pl.BlockSpec((BN, DC), lambda i, q: (i, 0)),
            pl.BlockSpec((BN, DC), lambda i, q: (nb + i, 0)),
            pl.BlockSpec((BN, DC), lambda i, q: (i, 0)),
            pl.BlockSpec((BN, DC), lambda i, q: (nb + i, 0)),
            pl.BlockSpec((h, DC), lambda i, q: (0, q)),
            pl.BlockSpec((h, DC), lambda i, q: (0, q)),
            pl.BlockSpec((1, DC), lambda i, q: (0, q)),
        ],
        out_specs=pl.BlockSpec((BN, DC), lambda i, q: (q * nb + i, 0)),
        out_shape=jax.ShapeDtypeStruct((4 * NP, DC), jnp.float32),
    )(x, agg1c, agg1c, deg2, deg2, w_self, w_neigh, b)


def _k2_body(h0, h1, h2, h3, a0, a1, a2, a3, d0_ref, d1_ref, ws_ref, wn_ref,
             b2_ref, wf1_ref, bf1_ref, wf2_ref, bf2_ref, o_ref):
    h = jnp.concatenate([h0[...], h1[...], h2[...], h3[...]], axis=1)
    agg = jnp.concatenate([a0[...], a1[...], a2[...], a3[...]], axis=1)
    deg = d0_ref[...][:, 0:1] + d1_ref[...][:, 0:1]
    recip = 1.0 / jnp.maximum(deg, 1.0)
    acc = jnp.dot(h.astype(jnp.bfloat16), ws_ref[...],
                  preferred_element_type=jnp.float32)
    acc = acc + jnp.dot(agg.astype(jnp.bfloat16), wn_ref[...],
                        preferred_element_type=jnp.float32) * recip
    h2v = jnp.maximum(acc + b2_ref[...], 0.0)
    z = jnp.maximum(
        jnp.dot(h2v.astype(jnp.bfloat16), wf1_ref[...],
                preferred_element_type=jnp.float32) + bf1_ref[...],
        0.0)
    o_ref[...] = (jnp.dot(z, wf2_ref[...], preferred_element_type=jnp.float32)
                  + bf2_ref[...])


def _layer2_head(h1c, agg2c, deg2, w_self, w_neigh, b2, w_fc1, b_fc1, w_fc2,
                 b_fc2):
    nb = NP // BN
    hh = w_self.shape[0]
    chunk_spec = lambda q: pl.BlockSpec((BN, DC), lambda i, q=q: (q * nb + i, 0))
    return pl.pallas_call(
        _k2_body,
        grid=(nb,),
        in_specs=(
            [chunk_spec(q) for q in range(4)]
            + [chunk_spec(q) for q in range(4)]
            + [
                pl.BlockSpec((BN, DC), lambda i: (i, 0)),
                pl.BlockSpec((BN, DC), lambda i: (nb + i, 0)),
                pl.BlockSpec((hh, hh), lambda i: (0, 0)),
                pl.BlockSpec((hh, hh), lambda i: (0, 0)),
                pl.BlockSpec((1, hh), lambda i: (0, 0)),
                pl.BlockSpec((hh, hh), lambda i: (0, 0)),
                pl.BlockSpec((1, hh), lambda i: (0, 0)),
                pl.BlockSpec((hh, 1), lambda i: (0, 0)),
                pl.BlockSpec((1, 1), lambda i: (0, 0)),
            ]
        ),
        out_specs=pl.BlockSpec((BN, 1), lambda i: (i, 0)),
        out_shape=jax.ShapeDtypeStruct((NP, 1), jnp.float32),
    )(*([h1c] * 4), *([agg2c] * 4), deg2, deg2, w_self, w_neigh, b2, w_fc1,
      b_fc1, w_fc2, b_fc2)


def kernel(x, edge_index, W_self1, W_neigh1, b1, W_self2, W_neigh2, b2,
           W_fc1, b_fc1, W_fc2, b_fc2):
    src = edge_index[0]
    dst = edge_index[1]
    xp = jnp.pad(x, ((0, NP - N), (0, 0)))
    xc = xp.reshape(NP, 2, DC).transpose(1, 0, 2).reshape(2 * NP, DC)

    src3d = jnp.pad(src, (0, E_PAD - E), constant_values=N).reshape(NROW, K, C)
    dst3d = jnp.pad(dst, (0, E_PAD - E), constant_values=N).reshape(NROW, K, C)
    zeros = jnp.zeros((RPT, DC), jnp.float32)
    bf = jnp.bfloat16
    agg1c, deg2 = _make_sc_agg(2, True)(xc, src3d, dst3d, zeros)
    h1c = _layer1(xp, agg1c, deg2, W_self1.astype(bf), W_neigh1.astype(bf),
                  b1.reshape(1, -1))
    (agg2c,) = _make_sc_agg(4, False)(h1c, src3d, dst3d, zeros)
    scores = _layer2_head(h1c, agg2c, deg2, W_self2.astype(bf),
                          W_neigh2.astype(bf), b2.reshape(1, -1),
                          W_fc1.astype(bf), b_fc1.reshape(1, -1),
                          W_fc2, b_fc2.reshape(1, 1))
    return scores[:N]
